# baseline (speedup 1.0000x reference)
import functools
import numpy as np
import jax
import jax.numpy as jnp
from jax import lax
from jax.experimental import pallas as pl
from jax.experimental.pallas import tpu as pltpu
from jax.experimental.pallas import tpu_sc as plsc

B = 16384
SEQ = 50
D = 64
NC, NS, L = 2, 16, 16
NW = NC * NS
B_BLK = 256
NBB = B // B_BLK
BLKS_PER_W = SEQ * NBB // NW
NVR = D // L


def _make_pe_const():
    position = np.arange(0, SEQ, dtype=np.float32)[:, None]
    div_term = np.exp(
        np.arange(0, D, 2, dtype=np.float32) * (-np.log(10000.0) / D))
    pe = np.zeros((SEQ, D), dtype=np.float32)
    pe[:, 0::2] = np.sin(position * div_term)
    pe[:, 1::2] = np.cos(position * div_term)
    return jnp.asarray(pe)


_MESH = plsc.VectorSubcoreMesh(core_axis_name="c", subcore_axis_name="s")


@functools.partial(
    pl.kernel,
    mesh=_MESH,
    compiler_params=pltpu.CompilerParams(
        use_tc_tiling_on_sc=False, needs_layout_passes=False),
    out_type=jax.ShapeDtypeStruct((SEQ, D, B), jnp.float32),
    scratch_types=[
        pltpu.VMEM((2, B_BLK), jnp.int32),
        pltpu.VMEM((2, B_BLK, D), jnp.float32),
        pltpu.VMEM((2, D, B_BLK + 1), jnp.float32),
        pltpu.VMEM((SEQ, D), jnp.float32),
    ]
    + [pltpu.SemaphoreType.DMA] * 4,
)
def _embed_pe(tok_hbm, table_hbm, pe_hbm, out_hbm, idx_v, rows_v, stage_v,
              pe_v, gsem0, gsem1, osem0, osem1):
    gsem = (gsem0, gsem1)
    osem = (osem0, osem1)
    wid = lax.axis_index("s") * NC + lax.axis_index("c")
    pltpu.sync_copy(pe_hbm, pe_v)

    iota = lax.iota(jnp.int32, L)
    row_idx = [jg * L + iota for jg in range(NVR)]
    zero_v = iota - iota

    def load_idx(c, bf):
        p = c // 2
        b0 = (2 * wid + (c % 2)) * B_BLK
        pltpu.sync_copy(tok_hbm.at[p, pl.ds(b0, B_BLK)], idx_v.at[bf])
        for k in range(B_BLK // L):
            sl = pl.ds(k * L, L)
            idx_v[bf, sl] = idx_v[bf, sl] * 2

    def start_gather(bf):
        pltpu.async_copy(table_hbm.at[idx_v.at[bf]], rows_v.at[bf], gsem[bf])

    def wait_gather(bf):
        pltpu.make_async_copy(
            table_hbm.at[idx_v.at[bf]], rows_v.at[bf], gsem[bf]).wait()

    def wait_out(bf):
        pltpu.make_async_copy(
            stage_v.at[bf, :, pl.ds(0, B_BLK)],
            out_hbm.at[0, :, pl.ds(0, B_BLK)], osem[bf]).wait()

    load_idx(0, 0)
    start_gather(0)

    def pair_body(i, carry):
        for par in range(2):
            c = 2 * i + par
            bf = par
            nbf = 1 - par

            @pl.when(c + 1 < BLKS_PER_W)
            def _issue_next():
                load_idx(c + 1, nbf)
                start_gather(nbf)

            wait_gather(bf)

            @pl.when(c >= 2)
            def _drain_stage():
                wait_out(bf)

            p = c // 2
            b0 = (2 * wid + par) * B_BLK
            pe_regs = [pe_v[p, pl.ds(jg * L, L)] for jg in range(NVR)]
            rows_b = rows_v.at[bf]
            stage_b = stage_v.at[bf]

            @plsc.parallel_loop(0, B_BLK, step=8, unroll=4)
            def s_body(s0):
                for ds_ in range(8):
                    s = s0 + ds_
                    col_idx = zero_v + s
                    for jg in range(NVR):
                        v = rows_b[s, pl.ds(jg * L, L)] + pe_regs[jg]
                        plsc.store_scatter(
                            stage_b, [row_idx[jg], col_idx], v)
            pltpu.async_copy(
                stage_v.at[bf, :, pl.ds(0, B_BLK)],
                out_hbm.at[p, :, pl.ds(b0, B_BLK)], osem[bf])
        return carry

    lax.fori_loop(0, BLKS_PER_W // 2, pair_body, 0)
    wait_out(0)
    wait_out(1)


def kernel(tokens, table):
    pe = _make_pe_const()
    tbl2 = jnp.pad(table, ((0, 0), (0, D))).reshape(2 * table.shape[0], D)
    tok_t = tokens.T.astype(jnp.int32)
    out_t = _embed_pe(tok_t, tbl2, pe)
    return jnp.transpose(out_t, (2, 0, 1))


# speedup vs baseline: 1.0874x; 1.0874x over previous
import functools
import numpy as np
import jax
import jax.numpy as jnp
from jax import lax
from jax.experimental import pallas as pl
from jax.experimental.pallas import tpu as pltpu
from jax.experimental.pallas import tpu_sc as plsc

B = 16384
SEQ = 50
D = 64
NC, NS, L = 2, 16, 16
NW = NC * NS
B_BLK = 256
NBB = B // B_BLK
BLKS_PER_W = SEQ * NBB // NW
NVR = D // L


def _make_pe_const():
    position = np.arange(0, SEQ, dtype=np.float32)[:, None]
    div_term = np.exp(
        np.arange(0, D, 2, dtype=np.float32) * (-np.log(10000.0) / D))
    pe = np.zeros((SEQ, D), dtype=np.float32)
    pe[:, 0::2] = np.sin(position * div_term)
    pe[:, 1::2] = np.cos(position * div_term)
    return jnp.asarray(pe)


_MESH = plsc.VectorSubcoreMesh(core_axis_name="c", subcore_axis_name="s")


@functools.partial(
    pl.kernel,
    mesh=_MESH,
    compiler_params=pltpu.CompilerParams(
        use_tc_tiling_on_sc=False, needs_layout_passes=False),
    out_type=jax.ShapeDtypeStruct((SEQ, D, B), jnp.float32),
    scratch_types=[
        pltpu.VMEM((2, B_BLK), jnp.int32),
        pltpu.VMEM((2, B_BLK, D), jnp.float32),
        pltpu.VMEM((2, D, B_BLK + 1), jnp.float32),
        pltpu.VMEM((SEQ, D), jnp.float32),
    ]
    + [pltpu.SemaphoreType.DMA] * 4,
)
def _embed_pe(tok_hbm, table_hbm, pe_hbm, out_hbm, idx_v, rows_v, stage_v,
              pe_v, gsem0, gsem1, osem0, osem1):
    gsem = (gsem0, gsem1)
    osem = (osem0, osem1)
    wid = lax.axis_index("s") * NC + lax.axis_index("c")
    pltpu.sync_copy(pe_hbm, pe_v)

    iota = lax.iota(jnp.int32, L)
    row_idx = [jg * L + iota for jg in range(NVR)]
    zero_v = iota - iota

    def load_idx(c, bf):
        p = c // 2
        b0 = (2 * wid + (c % 2)) * B_BLK
        pltpu.sync_copy(tok_hbm.at[p, pl.ds(b0, B_BLK)], idx_v.at[bf])
        for k in range(B_BLK // L):
            sl = pl.ds(k * L, L)
            idx_v[bf, sl] = idx_v[bf, sl] * 2

    def start_gather(bf):
        pltpu.async_copy(table_hbm.at[idx_v.at[bf]], rows_v.at[bf], gsem[bf])

    def wait_gather(bf):
        pltpu.make_async_copy(
            table_hbm.at[idx_v.at[bf]], rows_v.at[bf], gsem[bf]).wait()

    def wait_out(bf):
        pltpu.make_async_copy(
            stage_v.at[bf, :, pl.ds(0, B_BLK)],
            out_hbm.at[0, :, pl.ds(0, B_BLK)], osem[bf]).wait()

    load_idx(0, 0)
    start_gather(0)

    def pair_body(i, carry):
        for par in range(2):
            c = 2 * i + par
            bf = par
            nbf = 1 - par

            @pl.when(c + 1 < BLKS_PER_W)
            def _issue_next():
                load_idx(c + 1, nbf)
                start_gather(nbf)

            wait_gather(bf)

            @pl.when(c >= 2)
            def _drain_stage():
                wait_out(bf)

            p = c // 2
            b0 = (2 * wid + par) * B_BLK
            pe_regs = [pe_v[p, pl.ds(jg * L, L)] for jg in range(NVR)]
            rows_b = rows_v.at[bf]
            stage_b = stage_v.at[bf]

            @plsc.parallel_loop(0, B_BLK, step=8, unroll=2)
            def s_body(s0):
                for ds_ in range(8):
                    s = s0 + ds_
                    col_idx = zero_v + s
                    for jg in range(NVR):
                        v = rows_b[s, pl.ds(jg * L, L)] + pe_regs[jg]
                        plsc.store_scatter(
                            stage_b, [row_idx[jg], col_idx], v)
            pltpu.async_copy(
                stage_v.at[bf, :, pl.ds(0, B_BLK)],
                out_hbm.at[p, :, pl.ds(b0, B_BLK)], osem[bf])
        return carry

    lax.fori_loop(0, BLKS_PER_W // 2, pair_body, 0)
    wait_out(0)
    wait_out(1)


def kernel(tokens, table):
    pe = _make_pe_const()
    tbl2 = jnp.pad(table, ((0, 0), (0, D))).reshape(2 * table.shape[0], D)
    tok_t = tokens.T.astype(jnp.int32)
    out_t = _embed_pe(tok_t, tbl2, pe)
    return jnp.transpose(out_t, (2, 0, 1))
